# Initial kernel scaffold; baseline (speedup 1.0000x reference)
import jax
import jax.numpy as jnp
from jax.experimental import pallas as pl


def kernel(x, edges, W_l, W_r, b, gamma, beta, W_out, b_out):
    raise NotImplementedError("write your pallas kernel here")


# speedup vs baseline: 2.7849x; 2.7849x over previous
import functools

import jax
import jax.numpy as jnp
from jax import lax
from jax.experimental import pallas as pl
from jax.experimental.pallas import tpu as pltpu
from jax.experimental.pallas import tpu_sc as plsc

_N_NODES = 10000
_N_EDGES = 320000
_D = 128
_NC = 2
_NS = 16
_NWORK = _NC * _NS
_CH = 128
_EPW = 10240
_NCHUNK = _EPW // _CH
_AGG_ROWS = 10240
_RPT = _AGG_ROWS // _NS

_mesh = plsc.VectorSubcoreMesh(core_axis_name="c", subcore_axis_name="s")


@functools.partial(
    pl.kernel,
    out_type=jax.ShapeDtypeStruct((_NC, _AGG_ROWS, _D), jnp.float32),
    mesh=_mesh,
    scratch_types=[
        pltpu.VMEM((_NCHUNK, _CH), jnp.int32),
        pltpu.VMEM((_NCHUNK, _CH), jnp.int32),
        pltpu.VMEM((_CH, _D), jnp.float32),
        pltpu.VMEM_SHARED((_AGG_ROWS, _D), jnp.float32),
        pltpu.SemaphoreType.DMA,
    ],
)
def _sc_agg(h_hbm, srcs_hbm, dsts_hbm, zeros_hbm, out_hbm,
            src_v, dst_v, g_v, agg_sh, sem):
    cid = lax.axis_index("c")
    sid = lax.axis_index("s")
    wid = cid * _NS + sid
    pltpu.sync_copy(srcs_hbm.at[wid], src_v)
    pltpu.sync_copy(dsts_hbm.at[wid], dst_v)
    pltpu.sync_copy(zeros_hbm, agg_sh.at[pl.ds(sid * _RPT, _RPT)])
    plsc.subcore_barrier()

    def body(j, carry):
        pltpu.async_copy(h_hbm.at[src_v.at[j]], g_v, sem).wait()
        pltpu.sync_copy(g_v, agg_sh.at[dst_v.at[j]], add=True)
        return carry

    lax.fori_loop(0, _NCHUNK, body, 0)
    plsc.subcore_barrier()
    pltpu.sync_copy(agg_sh.at[pl.ds(sid * _RPT, _RPT)],
                    out_hbm.at[cid, pl.ds(sid * _RPT, _RPT)])


@functools.partial(
    pl.kernel,
    out_type=jax.ShapeDtypeStruct((_NC, _AGG_ROWS, _D), jnp.float32),
    mesh=_mesh,
    scratch_types=[
        pltpu.VMEM((_NCHUNK, _CH), jnp.int32),
        pltpu.VMEM((_CH, _D), jnp.float32),
        pltpu.VMEM_SHARED((_AGG_ROWS, _D), jnp.float32),
        pltpu.SemaphoreType.DMA,
    ],
)
def _sc_cnt(dsts_hbm, ones_hbm, zeros_hbm, out_hbm, dst_v, ones_v, cnt_sh, sem):
    cid = lax.axis_index("c")
    sid = lax.axis_index("s")
    wid = cid * _NS + sid
    pltpu.sync_copy(dsts_hbm.at[wid], dst_v)
    pltpu.sync_copy(ones_hbm, ones_v)
    pltpu.sync_copy(zeros_hbm, cnt_sh.at[pl.ds(sid * _RPT, _RPT)])
    plsc.subcore_barrier()

    def body(j, carry):
        pltpu.sync_copy(ones_v, cnt_sh.at[dst_v.at[j]], add=True)
        return carry

    lax.fori_loop(0, _NCHUNK, body, 0)
    plsc.subcore_barrier()
    pltpu.sync_copy(cnt_sh.at[pl.ds(sid * _RPT, _RPT)],
                    out_hbm.at[cid, pl.ds(sid * _RPT, _RPT)])


def _tc_block_body(h_ref, agg_ref, cnt_ref, wl_ref, wr_ref, b_ref,
                   ga_ref, be_ref, out_ref):
    h = h_ref[...]
    agg = agg_ref[0, :_N_NODES, :] + agg_ref[1, :_N_NODES, :]
    cnt = cnt_ref[0, :_N_NODES, :] + cnt_ref[1, :_N_NODES, :]
    c = jnp.maximum(cnt[:, 0:1], 1.0)
    mean = agg / c
    z = jnp.dot(mean, wl_ref[...], preferred_element_type=jnp.float32)
    z = z + jnp.dot(h, wr_ref[...], preferred_element_type=jnp.float32)
    z = z + b_ref[...]
    mu = jnp.mean(z, axis=0, keepdims=True)
    var = jnp.mean((z - mu) ** 2, axis=0, keepdims=True)
    zn = ga_ref[...] * (z - mu) * lax.rsqrt(var + 1e-5) + be_ref[...]
    e = jnp.where(zn > 0, zn, jnp.exp(jnp.minimum(zn, 0.0)) - 1.0)
    out_ref[...] = h + e


_tc_block = pl.pallas_call(
    _tc_block_body,
    out_shape=jax.ShapeDtypeStruct((_N_NODES, _D), jnp.float32),
)


def _tc_out_body(h_ref, w_ref, b_ref, out_ref):
    out_ref[...] = jnp.tanh(
        jnp.dot(h_ref[...], w_ref[...], preferred_element_type=jnp.float32)
        + b_ref[...])


_tc_out = pl.pallas_call(
    _tc_out_body,
    out_shape=jax.ShapeDtypeStruct((_N_NODES, _D), jnp.float32),
)


def kernel(x, edges, W_l, W_r, b, gamma, beta, W_out, b_out):
    f32 = jnp.float32
    pad = _NWORK * _EPW - _N_EDGES
    src_p = jnp.concatenate(
        [edges[0], jnp.zeros((pad,), jnp.int32)]).reshape(_NWORK, _NCHUNK, _CH)
    dst_p = jnp.concatenate(
        [edges[1], jnp.full((pad,), _N_NODES, jnp.int32)]).reshape(
            _NWORK, _NCHUNK, _CH)
    zeros_d = jnp.zeros((_RPT, _D), f32)
    ones_d = jnp.ones((_CH, _D), f32)

    cnt = _sc_cnt(dst_p, ones_d, zeros_d)
    h = x
    for i in range(7):
        agg = _sc_agg(h, src_p, dst_p, zeros_d)
        h = _tc_block(h, agg, cnt, W_l[i], W_r[i], b[i].reshape(1, _D),
                      gamma[i].reshape(1, _D), beta[i].reshape(1, _D))
    w_pad = jnp.pad(W_out, ((0, 0), (0, _D - W_out.shape[1])))
    b_pad = jnp.pad(b_out, (0, _D - b_out.shape[0])).reshape(1, _D)
    out = _tc_out(h, w_pad, b_pad)
    return out[:, :W_out.shape[1]]
